# baseline (speedup 1.0000x reference)
import functools

import jax
import jax.numpy as jnp
from jax import lax
from jax.experimental import pallas as pl
from jax.experimental.pallas import tpu as pltpu
from jax.experimental.pallas import tpu_sc as plsc

N = 10000
E = 320000
D = 128
D_EDGE = 16

_BE = 8000


def _edge_lin_body(ea_ref, we_ref, be_ref, out_ref):
    h = (
        jnp.dot(ea_ref[...], we_ref[...], preferred_element_type=jnp.float32)
        + be_ref[...]
    ).astype(jnp.bfloat16)
    out_ref[...] = pltpu.bitcast(h, jnp.int32)


def _edge_linear(edge_attr, We, be):
    return pl.pallas_call(
        _edge_lin_body,
        grid=(E // _BE,),
        in_specs=[
            pl.BlockSpec((_BE, D_EDGE), lambda i: (i, 0)),
            pl.BlockSpec((D_EDGE, D), lambda i: (0, 0)),
            pl.BlockSpec((1, D), lambda i: (0, 0)),
        ],
        out_specs=pl.BlockSpec((_BE // 2, D), lambda i: (i, 0)),
        out_shape=jax.ShapeDtypeStruct((E // 2, D), jnp.int32),
    )(edge_attr, We, be.reshape(1, D))


_NW = 32
_EPW = E // _NW
_K = 40
_NCHUNK = _EPW // _K
_NG = _NCHUNK // 2
_RB = 16
_RPT = 624


def _sc_aggregate(x, srcw, dstw, e2):
    mesh = plsc.VectorSubcoreMesh(core_axis_name="c", subcore_axis_name="s")

    @functools.partial(
        pl.kernel,
        mesh=mesh,
        compiler_params=pltpu.CompilerParams(needs_layout_passes=False),
        out_type=jax.ShapeDtypeStruct((2, N, D), jnp.float32),
        scratch_types=[
            pltpu.VMEM((_EPW,), jnp.int32),
            pltpu.VMEM((_EPW,), jnp.int32),
            pltpu.VMEM((_K, D), jnp.float32),
            pltpu.VMEM((_K, D), jnp.float32),
            pltpu.VMEM((_K, D), jnp.int32),
            pltpu.VMEM((_K, D), jnp.int32),
            pltpu.VMEM((_K, D), jnp.float32),
            pltpu.VMEM((_K, D), jnp.float32),
            pltpu.VMEM_SHARED((N, D), jnp.float32),
            pltpu.SemaphoreType.DMA,
            pltpu.SemaphoreType.DMA,
            pltpu.SemaphoreType.DMA,
            pltpu.SemaphoreType.DMA,
            pltpu.SemaphoreType.DMA,
            pltpu.SemaphoreType.DMA,
            pltpu.SemaphoreType.DMA,
        ],
    )
    def k(x_hbm, src_hbm, dst_hbm, e_hbm, out_hbm,
          src_v, dst_v, xj0, xj1, eg0, eg1, m0, m1, acc_sh,
          gsem0, gsem1, ssem0, ssem1, esemA, esemB, isem):
        c = lax.axis_index("c")
        s = lax.axis_index("s")
        w = c * 16 + s
        ebase = w * (_EPW // 2)
        xj = (xj0, xj1)
        eg = (eg0, eg1)
        mb = (m0, m1)
        gsem = (gsem0, gsem1)
        ssem = (ssem0, ssem1)
        esem = (esemA, esemB)

        nper = _RPT // _RB + jnp.where(s == 15, 1, 0)
        zero16 = jnp.zeros((16,), jnp.float32)

        def zrow(i, carry):
            for j in range(8):
                m0[i, pl.ds(j * 16, 16)] = zero16
            return carry

        lax.fori_loop(0, _RB, zrow, 0)

        def zcopy(t, carry):
            pltpu.sync_copy(m0.at[pl.ds(0, _RB)],
                            acc_sh.at[pl.ds(s * _RPT + t * _RB, _RB)])
            return carry

        lax.fori_loop(0, nper, zcopy, 0)

        pltpu.async_copy(src_hbm.at[w], src_v, isem).wait()
        pltpu.async_copy(dst_hbm.at[w], dst_v, isem).wait()
        plsc.subcore_barrier()

        def issue_g(g, b):
            pltpu.async_copy(x_hbm.at[src_v.at[pl.ds(g * _K, _K)]], xj[b],
                             gsem[b])

        def issue_g_maybe(g, b):
            @pl.when(g < _NCHUNK)
            def _():
                issue_g(g, b)

        def wait_g(b):
            pltpu.make_async_copy(x_hbm.at[pl.ds(0, _K)], xj[b],
                                  gsem[b]).wait()

        def issue_e(G, gb):
            pltpu.async_copy(e_hbm.at[pl.ds(ebase + G * _K, _K)], eg[gb],
                             esem[gb])

        def issue_e_maybe(G, gb):
            @pl.when(G < _NG)
            def _():
                issue_e(G, gb)

        def wait_e(gb):
            pltpu.make_async_copy(e_hbm.at[pl.ds(0, _K)], eg[gb],
                                  esem[gb]).wait()

        def compute(b, gb, p):
            xb, egb, mbb = xj[b], eg[gb], mb[b]

            def pair(i, cc):
                t0 = 2 * i
                t1 = 2 * i + 1
                for u, t in ((0, t0), (1, t1)):
                    re = p * (_K // 2) + t
                    r0 = 2 * t
                    r1 = 2 * t + 1
                    for j in range(8):
                        sl = pl.ds(j * 16, 16)
                        ev = plsc.bitcast(egb[re, sl], jnp.bfloat16)
                        elo, ehi = plsc.unpack(
                            ev, format=plsc.PackFormat.INTERLEAVED)
                        mbb[r0, sl] = jnp.maximum(xb[r0, sl] + elo, 0.0)
                        mbb[r1, sl] = jnp.maximum(xb[r1, sl] + ehi, 0.0)
                return cc

            lax.fori_loop(0, _K // 4, pair, 0)

        def scatter(g, b):
            pltpu.async_copy(mb[b], acc_sh.at[dst_v.at[pl.ds(g * _K, _K)]],
                             ssem[b], add=True)

        def wait_s(b):
            pltpu.make_async_copy(out_hbm.at[0, pl.ds(0, _K)], mb[b],
                                  ssem[b]).wait()

        issue_e(0, 0)
        issue_e(1, 1)
        issue_g(0, 0)
        issue_g(1, 1)
        wait_g(0)
        wait_e(0)
        compute(0, 0, 0)
        scatter(0, 0)
        issue_g(2, 0)
        wait_g(1)
        compute(1, 0, 1)
        scatter(1, 1)
        issue_g(3, 1)
        issue_e(2, 0)

        def body(i, carry):
            c0 = 2 + 4 * i
            wait_g(0)
            wait_e(1)
            wait_s(0)
            compute(0, 1, 0)
            scatter(c0, 0)
            issue_g_maybe(c0 + 2, 0)
            wait_g(1)
            wait_s(1)
            compute(1, 1, 1)
            scatter(c0 + 1, 1)
            issue_g_maybe(c0 + 3, 1)
            issue_e_maybe(3 + 2 * i, 1)
            wait_g(0)
            wait_e(0)
            wait_s(0)
            compute(0, 0, 0)
            scatter(c0 + 2, 0)
            issue_g_maybe(c0 + 4, 0)
            wait_g(1)
            wait_s(1)
            compute(1, 0, 1)
            scatter(c0 + 3, 1)
            issue_g_maybe(c0 + 5, 1)
            issue_e_maybe(4 + 2 * i, 0)
            return carry

        lax.fori_loop(0, (_NCHUNK - 2) // 4, body, 0)

        wait_s(0)
        wait_s(1)
        plsc.subcore_barrier()

        def drain(t, carry):
            off = s * _RPT + t * _RB
            pltpu.sync_copy(acc_sh.at[pl.ds(off, _RB)], m0.at[pl.ds(0, _RB)])
            pltpu.sync_copy(m0.at[pl.ds(0, _RB)], out_hbm.at[c, pl.ds(off, _RB)])
            return carry

        lax.fori_loop(0, nper, drain, 0)

    return k(x, srcw, dstw, e2)


_BN = 2000


def _mlp_body(x_ref, agg_ref, w1_ref, b1_ref, w2_ref, b2_ref, lg_ref, lb_ref,
              out_ref):
    xb = x_ref[...]
    h = xb + agg_ref[0] + agg_ref[1]
    h1 = jnp.maximum(
        jnp.dot(h, w1_ref[...], preferred_element_type=jnp.float32)
        + b1_ref[...], 0.0)
    h2 = (jnp.dot(h1, w2_ref[...], preferred_element_type=jnp.float32)
          + b2_ref[...])
    mu = jnp.mean(h2, axis=-1, keepdims=True)
    var = jnp.mean(jnp.square(h2 - mu), axis=-1, keepdims=True)
    hn = (h2 - mu) * lax.rsqrt(var + 1e-5) * lg_ref[...] + lb_ref[...]
    z = hn + xb
    out_ref[...] = jnp.where(z > 0, z, jnp.exp(jnp.minimum(z, 0.0)) - 1.0)


def _mlp(x, agg2, W1f, b1f, W2, b2, ln_gamma, ln_beta):
    return pl.pallas_call(
        _mlp_body,
        grid=(N // _BN,),
        in_specs=[
            pl.BlockSpec((_BN, D), lambda i: (i, 0)),
            pl.BlockSpec((2, _BN, D), lambda i: (0, i, 0)),
            pl.BlockSpec((D, D), lambda i: (0, 0)),
            pl.BlockSpec((1, D), lambda i: (0, 0)),
            pl.BlockSpec((D, D), lambda i: (0, 0)),
            pl.BlockSpec((1, D), lambda i: (0, 0)),
            pl.BlockSpec((1, D), lambda i: (0, 0)),
            pl.BlockSpec((1, D), lambda i: (0, 0)),
        ],
        out_specs=pl.BlockSpec((_BN, D), lambda i: (i, 0)),
        out_shape=jax.ShapeDtypeStruct((N, D), jnp.float32),
    )(x, agg2, W1f, b1f.reshape(1, D), W2, b2.reshape(1, D),
      ln_gamma.reshape(1, D), ln_beta.reshape(1, D))


def kernel(x, edge_index, edge_attr, We, be, W1, b1, bn_gamma, bn_beta,
           bn_mean, bn_var, W2, b2, ln_gamma, ln_beta):
    srcw = edge_index[0].reshape(_NW, _EPW)
    dstw = edge_index[1].reshape(_NW, _EPW)
    e2 = _edge_linear(edge_attr, We, be)
    agg2 = _sc_aggregate(x, srcw, dstw, e2)
    scale = bn_gamma * lax.rsqrt(bn_var + 1e-5)
    W1f = W1 * scale[None, :]
    b1f = (b1 - bn_mean) * scale + bn_beta
    return _mlp(x, agg2, W1f, b1f, W2, b2, ln_gamma, ln_beta)


# speedup vs baseline: 1.1013x; 1.1013x over previous
import functools

import jax
import jax.numpy as jnp
from jax import lax
from jax.experimental import pallas as pl
from jax.experimental.pallas import tpu as pltpu
from jax.experimental.pallas import tpu_sc as plsc

N = 10000
E = 320000
EH = E // 2
D = 128
D_EDGE = 16

_BE = 8000


def _edge_lin_body(ea_ref, we_ref, be_ref, out_ref):
    out_ref[...] = (
        jnp.dot(ea_ref[...], we_ref[...], preferred_element_type=jnp.float32)
        + be_ref[...]
    )


def _edge_linear(edge_attr, We, be):
    return pl.pallas_call(
        _edge_lin_body,
        grid=(EH // _BE,),
        in_specs=[
            pl.BlockSpec((_BE, D_EDGE), lambda i: (i, 0)),
            pl.BlockSpec((D_EDGE, D), lambda i: (0, 0)),
            pl.BlockSpec((1, D), lambda i: (0, 0)),
        ],
        out_specs=pl.BlockSpec((_BE, D), lambda i: (i, 0)),
        out_shape=jax.ShapeDtypeStruct((EH, D), jnp.float32),
    )(edge_attr, We, be.reshape(1, D))


_NW = 32
_EPW = EH // _NW
_K = 40
_NCHUNK = _EPW // _K
_RB = 16
_RPT = 624


def _sc_aggregate(x, srcw, dstw, e):
    mesh = plsc.VectorSubcoreMesh(core_axis_name="c", subcore_axis_name="s")

    @functools.partial(
        pl.kernel,
        mesh=mesh,
        out_type=jax.ShapeDtypeStruct((2, N, D), jnp.float32),
        scratch_types=[
            pltpu.VMEM((_EPW,), jnp.int32),
            pltpu.VMEM((_EPW,), jnp.int32),
            pltpu.VMEM((_K, D), jnp.float32),
            pltpu.VMEM((_K, D), jnp.float32),
            pltpu.VMEM((_K, D), jnp.float32),
            pltpu.VMEM((_K, D), jnp.float32),
            pltpu.VMEM((_K, D), jnp.float32),
            pltpu.VMEM((_K, D), jnp.float32),
            pltpu.VMEM_SHARED((N, D), jnp.float32),
            pltpu.SemaphoreType.DMA,
            pltpu.SemaphoreType.DMA,
            pltpu.SemaphoreType.DMA,
            pltpu.SemaphoreType.DMA,
            pltpu.SemaphoreType.DMA,
        ],
    )
    def k(x_hbm, src_hbm, dst_hbm, e_hbm, out_hbm,
          src_v, dst_v, xj0, xj1, e0, e1, m0, m1, acc_sh,
          gsem0, gsem1, ssem0, ssem1, isem):
        c = lax.axis_index("c")
        s = lax.axis_index("s")
        w = c * 16 + s
        base0 = w * _EPW
        xj = (xj0, xj1)
        eb = (e0, e1)
        mb = (m0, m1)
        gsem = (gsem0, gsem1)
        ssem = (ssem0, ssem1)

        nper = _RPT // _RB + jnp.where(s == 15, 1, 0)
        zero16 = jnp.zeros((16,), jnp.float32)

        def zrow(i, carry):
            for j in range(8):
                m0[i, pl.ds(j * 16, 16)] = zero16
            return carry

        lax.fori_loop(0, _RB, zrow, 0)

        def zcopy(t, carry):
            pltpu.sync_copy(m0.at[pl.ds(0, _RB)],
                            acc_sh.at[pl.ds(s * _RPT + t * _RB, _RB)])
            return carry

        lax.fori_loop(0, nper, zcopy, 0)

        pltpu.async_copy(src_hbm.at[w], src_v, isem).wait()
        pltpu.async_copy(dst_hbm.at[w], dst_v, isem).wait()
        plsc.subcore_barrier()

        def issue(g, b):
            pltpu.async_copy(x_hbm.at[src_v.at[pl.ds(g * _K, _K)]], xj[b],
                             gsem[b])
            pltpu.async_copy(e_hbm.at[pl.ds(base0 + g * _K, _K)], eb[b],
                             gsem[b])

        def wait_gather(b):
            pltpu.make_async_copy(e_hbm.at[pl.ds(0, _K)], eb[b],
                                  gsem[b]).wait()
            pltpu.make_async_copy(e_hbm.at[pl.ds(0, _K)], xj[b],
                                  gsem[b]).wait()

        def compute(b):
            xb, ebb, mbb = xj[b], eb[b], mb[b]

            def row(i, cc):
                for u in range(4):
                    r = 4 * i + u
                    for j in range(8):
                        sl = pl.ds(j * 16, 16)
                        mbb[r, sl] = jnp.maximum(xb[r, sl] + ebb[r, sl], 0.0)
                return cc

            lax.fori_loop(0, _K // 4, row, 0)

        def scatter(g, b):
            pltpu.async_copy(mb[b], acc_sh.at[dst_v.at[pl.ds(g * _K, _K)]],
                             ssem[b], add=True)

        def wait_scatter(b):
            pltpu.make_async_copy(e_hbm.at[pl.ds(0, _K)], mb[b],
                                  ssem[b]).wait()

        issue(0, 0)
        issue(1, 1)
        for g0 in (0, 1):
            wait_gather(g0)
            compute(g0)
            scatter(g0, g0)
            issue(g0 + 2, g0)

        def body(i, carry):
            t = 2 + 2 * i
            for b in (0, 1):
                g = t + b
                wait_gather(b)
                wait_scatter(b)
                compute(b)
                scatter(g, b)
                issue(g + 2, b)
            return carry

        lax.fori_loop(0, (_NCHUNK - 5) // 2, body, 0)

        for g, b in ((_NCHUNK - 3, 0), (_NCHUNK - 2, 1), (_NCHUNK - 1, 0)):
            wait_gather(b)
            wait_scatter(b)
            compute(b)
            scatter(g, b)
            if g == _NCHUNK - 3:
                issue(_NCHUNK - 1, 0)
        wait_scatter(1)
        wait_scatter(0)
        plsc.subcore_barrier()

        def drain(t, carry):
            off = s * _RPT + t * _RB
            pltpu.sync_copy(acc_sh.at[pl.ds(off, _RB)], m0.at[pl.ds(0, _RB)])
            pltpu.sync_copy(m0.at[pl.ds(0, _RB)], out_hbm.at[c, pl.ds(off, _RB)])
            return carry

        lax.fori_loop(0, nper, drain, 0)

    return k(x, srcw, dstw, e)


_BN = 2000


def _mlp_body(x_ref, aa_ref, ab_ref, w1_ref, b1_ref, w2_ref, b2_ref, lg_ref,
              lb_ref, out_ref):
    xb = x_ref[...]
    h = xb + (aa_ref[0] + aa_ref[1]) + (ab_ref[0] + ab_ref[1])
    h1 = jnp.maximum(
        jnp.dot(h, w1_ref[...], preferred_element_type=jnp.float32)
        + b1_ref[...], 0.0)
    h2 = (jnp.dot(h1, w2_ref[...], preferred_element_type=jnp.float32)
          + b2_ref[...])
    mu = jnp.mean(h2, axis=-1, keepdims=True)
    var = jnp.mean(jnp.square(h2 - mu), axis=-1, keepdims=True)
    hn = (h2 - mu) * lax.rsqrt(var + 1e-5) * lg_ref[...] + lb_ref[...]
    z = hn + xb
    out_ref[...] = jnp.where(z > 0, z, jnp.exp(jnp.minimum(z, 0.0)) - 1.0)


def _mlp(x, agg_a, agg_b, W1f, b1f, W2, b2, ln_gamma, ln_beta):
    return pl.pallas_call(
        _mlp_body,
        grid=(N // _BN,),
        in_specs=[
            pl.BlockSpec((_BN, D), lambda i: (i, 0)),
            pl.BlockSpec((2, _BN, D), lambda i: (0, i, 0)),
            pl.BlockSpec((2, _BN, D), lambda i: (0, i, 0)),
            pl.BlockSpec((D, D), lambda i: (0, 0)),
            pl.BlockSpec((1, D), lambda i: (0, 0)),
            pl.BlockSpec((D, D), lambda i: (0, 0)),
            pl.BlockSpec((1, D), lambda i: (0, 0)),
            pl.BlockSpec((1, D), lambda i: (0, 0)),
            pl.BlockSpec((1, D), lambda i: (0, 0)),
        ],
        out_specs=pl.BlockSpec((_BN, D), lambda i: (i, 0)),
        out_shape=jax.ShapeDtypeStruct((N, D), jnp.float32),
    )(x, agg_a, agg_b, W1f, b1f.reshape(1, D), W2, b2.reshape(1, D),
      ln_gamma.reshape(1, D), ln_beta.reshape(1, D))


def kernel(x, edge_index, edge_attr, We, be, W1, b1, bn_gamma, bn_beta,
           bn_mean, bn_var, W2, b2, ln_gamma, ln_beta):
    src_a = edge_index[0, :EH].reshape(_NW, _EPW)
    dst_a = edge_index[1, :EH].reshape(_NW, _EPW)
    src_b = edge_index[0, EH:].reshape(_NW, _EPW)
    dst_b = edge_index[1, EH:].reshape(_NW, _EPW)
    e_a = _edge_linear(edge_attr[:EH], We, be)
    e_b = _edge_linear(edge_attr[EH:], We, be)
    agg_a = _sc_aggregate(x, src_a, dst_a, e_a)
    agg_b = _sc_aggregate(x, src_b, dst_b, e_b)
    scale = bn_gamma * lax.rsqrt(bn_var + 1e-5)
    W1f = W1 * scale[None, :]
    b1f = (b1 - bn_mean) * scale + bn_beta
    return _mlp(x, agg_a, agg_b, W1f, b1f, W2, b2, ln_gamma, ln_beta)
